# baseline (speedup 1.0000x reference)
import jax
import jax.numpy as jnp
from jax import lax
from jax.experimental import pallas as pl
from jax.experimental.pallas import tpu as pltpu, tpu_sc as plsc

_A = 50000
_TA = _A // 8
_CPLANE = 12800000

_ABLK = 1024
_NBLK = 38
_A_TC = _ABLK * _NBLK
_A_SC = _A - _A_TC

_INFO = plsc.get_sparse_core_info()
_NC = _INFO.num_cores
_NS = _INFO.num_subcores
_NW = _NC * _NS

_CHUNK = 344
_NV = 22
_PAD = _NV * 16
_MAIN_A = _CHUNK * _NW
_TAIL_A = _A_SC - _MAIN_A
_NVT = _TAIL_A // 16


def _start_offsets(a_vec, koff):
    return (a_vec >> 3) * 2048 + (a_vec & 7) * 128 + koff


def _sc_body(w_hbm, idx_hbm, out_hbm, idx_v, widx_v, buf_v, widx_t, buf_t, sem):
    c_ = lax.axis_index("c")
    s_ = lax.axis_index("s")
    wid = s_ * _NC + c_

    pltpu.sync_copy(idx_hbm, idx_v)
    k0 = idx_v[...][0]
    koff = (k0 >> 7) * 1024 + (k0 & 127)
    lanes = lax.iota(jnp.int32, 16)
    off0 = _start_offsets(_A_TC + wid * _CHUNK + lanes, koff)

    gathers = []
    for cc in range(3):
        def build(m, off, cc=cc):
            widx_v[pl.ds(cc * _PAD + m * 16, 16)] = off
            return off + 4096

        lax.fori_loop(0, _NV, build, off0 + cc * _CPLANE)
        gathers.append(pltpu.async_copy(
            w_hbm.at[widx_v.at[pl.ds(cc * _PAD, _PAD)]],
            buf_v.at[pl.ds(cc * _PAD, _PAD)], sem))

    a0 = wid * _CHUNK
    for cc in range(3):
        gathers[cc].wait()
        pltpu.sync_copy(buf_v.at[pl.ds(cc * _PAD, _CHUNK)],
                        out_hbm.at[pl.ds(cc * _A_SC + a0, _CHUNK)])

    @pl.when(wid == 0)
    def _():
        off0t = _start_offsets(_A_TC + _MAIN_A + lanes, koff)
        tails = []
        for cc in range(3):
            def buildt(m, off, cc=cc):
                widx_t[pl.ds(cc * _TAIL_A + m * 16, 16)] = off
                return off + 4096

            lax.fori_loop(0, _NVT, buildt, off0t + cc * _CPLANE)
            tails.append(pltpu.async_copy(
                w_hbm.at[widx_t.at[pl.ds(cc * _TAIL_A, _TAIL_A)]],
                buf_t.at[pl.ds(cc * _TAIL_A, _TAIL_A)], sem))
        for cc in range(3):
            tails[cc].wait()
            pltpu.sync_copy(buf_t.at[pl.ds(cc * _TAIL_A, _TAIL_A)],
                            out_hbm.at[pl.ds(cc * _A_SC + _MAIN_A, _TAIL_A)])


_sc_fetch = pl.kernel(
    _sc_body,
    out_type=jax.ShapeDtypeStruct((3 * _A_SC,), jnp.float32),
    mesh=plsc.VectorSubcoreMesh(core_axis_name="c", subcore_axis_name="s"),
    scratch_types=[
        pltpu.VMEM((16,), jnp.int32),
        pltpu.VMEM((3 * _PAD,), jnp.int32),
        pltpu.VMEM((3 * _PAD,), jnp.float32),
        pltpu.VMEM((3 * _TAIL_A,), jnp.int32),
        pltpu.VMEM((3 * _TAIL_A,), jnp.float32),
        pltpu.SemaphoreType.DMA,
    ],
)


def _tc_body(s_ref, x_ref, o_ref):
    k1 = s_ref[0] % 128
    lane = lax.broadcasted_iota(jnp.int32, (1, 1, 128), 2)
    x = x_ref[...]
    o_ref[...] = jnp.sum(jnp.where(lane == k1, x, 0.0), axis=2)


def _tc_fetch(tv, conf_arr):
    grid_spec = pltpu.PrefetchScalarGridSpec(
        num_scalar_prefetch=1,
        grid=(_NBLK,),
        in_specs=[pl.BlockSpec((3, _ABLK, 128),
                               lambda i, s: (0, i, s[0] // 128))],
        out_specs=pl.BlockSpec((3, _ABLK), lambda i, s: (0, i)),
    )
    return pl.pallas_call(
        _tc_body,
        grid_spec=grid_spec,
        out_shape=jax.ShapeDtypeStruct((3, _A_TC), jnp.float32),
    )(conf_arr, tv)


@jax.jit
def kernel(conformational_space, conformation):
    conf = jnp.asarray(conformation, dtype=jnp.int32)
    tv = conformational_space.transpose(2, 1, 0)
    w = (tv.reshape(3, _TA, 8, 2, 128)
         .transpose(0, 1, 3, 2, 4)
         .reshape(-1))
    idx = jnp.full((16,), conf, dtype=jnp.int32)
    o_sc = _sc_fetch(w, idx)
    o_tc = _tc_fetch(tv, conf.reshape(1))
    o = jnp.concatenate([o_tc, o_sc.reshape(3, _A_SC)], axis=1)
    return o.transpose(1, 0)
